# baseline (speedup 1.0000x reference)
import functools

import jax
import jax.numpy as jnp
from jax import lax
from jax.experimental import pallas as pl
from jax.experimental.pallas import tpu as pltpu
from jax.experimental.pallas import tpu_sc as plsc

D_MODEL = 64
IDX_W = 128
K = 8
GROUP = K * IDX_W


@functools.lru_cache(maxsize=None)
def _make_gather(ntot: int, vocab: int):
    info = plsc.get_sparse_core_info()
    nc, ns = info.num_cores, info.num_subcores
    nw = nc * ns
    per_w = ntot // nw
    assert per_w * nw == ntot and per_w % GROUP == 0
    groups = per_w // GROUP
    mesh = plsc.VectorSubcoreMesh(core_axis_name="c", subcore_axis_name="s")

    @functools.partial(
        pl.kernel,
        out_type=jax.ShapeDtypeStruct((ntot, D_MODEL), jnp.float32),
        mesh=mesh,
        compiler_params=pltpu.CompilerParams(use_tc_tiling_on_sc=False),
        scratch_types=[
            pltpu.VMEM((K, IDX_W), jnp.int32),
            pltpu.VMEM((GROUP, D_MODEL), jnp.float32),
            pltpu.SemaphoreType.DMA,
        ],
    )
    def gather_k(idx_hbm, table_hbm, out_hbm, idx_v, rows_v, sem):
        wid = lax.axis_index("s") * nc + lax.axis_index("c")
        base = wid * per_w

        def group_body(g, carry):
            off = base + g * GROUP
            row = pl.multiple_of(off // IDX_W, 8)
            pltpu.sync_copy(idx_hbm.at[pl.ds(row, K)], idx_v)
            copies = [
                pltpu.async_copy(
                    table_hbm.at[idx_v.at[j]],
                    rows_v.at[pl.ds(j * IDX_W, IDX_W)],
                    sem,
                )
                for j in range(K)
            ]
            for c in copies:
                c.wait()
            pltpu.sync_copy(rows_v, out_hbm.at[pl.ds(off, GROUP)])
            return carry

        lax.fori_loop(0, groups, group_body, 0, unroll=False)

    return gather_k


def _mm_body(c_ref, w_ref, o_ref):
    o_ref[...] = lax.dot_general(
        c_ref[...], w_ref[...],
        dimension_numbers=(((1,), (1,)), ((), ())),
        preferred_element_type=jnp.float32,
    )


def _cond_proj(condition, w_cond):
    b = condition.shape[0]
    return pl.pallas_call(
        _mm_body,
        out_shape=jax.ShapeDtypeStruct((b, w_cond.shape[0]), jnp.float32),
    )(condition, w_cond)


def kernel(x, condition, table, W_cond):
    b, l = x.shape
    ntot = b * l
    idx = x.reshape(ntot // IDX_W, IDX_W).astype(jnp.int32)
    gather_k = _make_gather(ntot, table.shape[0])
    x_emb = gather_k(idx, table).reshape(b, l, D_MODEL)
    cond_emb = _cond_proj(condition, W_cond)
    return (x_emb, cond_emb)


# speedup vs baseline: 1.0139x; 1.0139x over previous
import functools

import jax
import jax.numpy as jnp
from jax import lax
from jax.experimental import pallas as pl
from jax.experimental.pallas import tpu as pltpu
from jax.experimental.pallas import tpu_sc as plsc

D_MODEL = 64
IDX_W = 128
BLK = 1024
HALF = BLK // 2
KH = HALF // IDX_W


@functools.lru_cache(maxsize=None)
def _make_gather(ntot: int):
    info = plsc.get_sparse_core_info()
    nc, ns = info.num_cores, info.num_subcores
    nw = nc * ns
    per_w = ntot // nw
    assert per_w * nw == ntot and per_w % BLK == 0
    nblk = per_w // BLK
    rows_per_blk = BLK // IDX_W
    n_idx_rows = ntot // IDX_W
    mesh = plsc.VectorSubcoreMesh(core_axis_name="c", subcore_axis_name="s")

    @functools.partial(
        pl.kernel,
        out_type=jax.ShapeDtypeStruct((ntot, D_MODEL), jnp.float32),
        mesh=mesh,
        compiler_params=pltpu.CompilerParams(use_tc_tiling_on_sc=False),
        scratch_types=[
            pltpu.VMEM((2, rows_per_blk, IDX_W), jnp.int32),
            pltpu.VMEM((2, HALF, D_MODEL), jnp.float32),
            pltpu.SemaphoreType.DMA,
            pltpu.SemaphoreType.DMA,
            pltpu.SemaphoreType.DMA,
            pltpu.SemaphoreType.DMA,
        ],
    )
    def gather_k(idx_hbm, table_hbm, out_hbm, idx_v, rows_v, gsem, ssem0,
                 ssem1, isem):
        wid = lax.axis_index("s") * nc + lax.axis_index("c")
        base = wid * per_w
        base_row = wid * (per_w // IDX_W)
        ssems = (ssem0, ssem1)

        def idx_fetch(b):
            row = lax.min(base_row + b * rows_per_blk,
                          n_idx_rows - rows_per_blk)
            row = pl.multiple_of(row, 8)
            return pltpu.make_async_copy(
                idx_hbm.at[pl.ds(row, rows_per_blk)], idx_v.at[b % 2], isem)

        def store_desc(p, off):
            return pltpu.make_async_copy(
                rows_v.at[p], out_hbm.at[pl.ds(off, HALF)], ssems[p])

        def half_iter(b, half, drain):
            p = half
            off = base + b * BLK + half * HALF
            if drain:
                store_desc(p, off).wait()
            copies = [
                pltpu.async_copy(
                    table_hbm.at[idx_v.at[b % 2].at[half * KH + jj]],
                    rows_v.at[p].at[pl.ds(jj * IDX_W, IDX_W)],
                    gsem)
                for jj in range(KH)
            ]
            if half == 0:
                idx_fetch(b + 1).start()
            for c in copies:
                c.wait()
            store_desc(p, off).start()

        idx_fetch(0).start()
        idx_fetch(0).wait()
        half_iter(0, 0, drain=False)
        half_iter(0, 1, drain=False)

        def body(b, carry):
            idx_fetch(b).wait()
            half_iter(b, 0, drain=True)
            half_iter(b, 1, drain=True)
            return carry

        lax.fori_loop(1, nblk, body, 0, unroll=False)

        idx_fetch(nblk).wait()
        store_desc(0, base + (nblk - 1) * BLK).wait()
        store_desc(1, base + (nblk - 1) * BLK + HALF).wait()

    return gather_k


def _mm_body(c_ref, w_ref, o_ref):
    o_ref[...] = lax.dot_general(
        c_ref[...], w_ref[...],
        dimension_numbers=(((1,), (1,)), ((), ())),
        preferred_element_type=jnp.float32,
    )


def _cond_proj(condition, w_cond):
    b = condition.shape[0]
    return pl.pallas_call(
        _mm_body,
        out_shape=jax.ShapeDtypeStruct((b, w_cond.shape[0]), jnp.float32),
    )(condition, w_cond)


def kernel(x, condition, table, W_cond):
    b, l = x.shape
    ntot = b * l
    idx = x.reshape(ntot // IDX_W, IDX_W).astype(jnp.int32)
    gather_k = _make_gather(ntot)
    x_emb = gather_k(idx, table).reshape(b, l, D_MODEL)
    cond_emb = _cond_proj(condition, W_cond)
    return (x_emb, cond_emb)
